# Initial kernel scaffold; baseline (speedup 1.0000x reference)
import jax
import jax.numpy as jnp
from jax.experimental import pallas as pl


def kernel(x0, edge_index0, edge_attr0, batch0, x1, edge_index1, edge_attr1, batch1, params):
    raise NotImplementedError("write your pallas kernel here")


# speedup vs baseline: 2.8809x; 2.8809x over previous
import functools
import jax
import jax.numpy as jnp
from jax import lax
from jax.experimental import pallas as pl
from jax.experimental.pallas import tpu as pltpu
from jax.experimental.pallas import tpu_sc as plsc

N = 10000
E = 160000
B = 400
EMB = 300
LAYERS = 5
TEMP = 0.04

NP = 10240
W = 384
WB = 128
NC = 2
NS = 16
RPT = NP // NS
ONE_COL = 304
BP = B + 16

def _mesh():
    return plsc.VectorSubcoreMesh(core_axis_name="c", subcore_axis_name="s")


def _sc_spmm(table3, src, src2, dst, zeros_blk):
    ch_a, nch_a = 80, (E // NS) // 80
    ch_b, nch_b = 40, (E // (NC * NS)) // 40

    @functools.partial(
        pl.kernel,
        out_type=jax.ShapeDtypeStruct((4 * NP, WB), jnp.float32),
        mesh=_mesh(),
        scratch_types=[
            pltpu.VMEM_SHARED((NP, WB), jnp.float32),
            pltpu.VMEM((ch_a,), jnp.int32),
            pltpu.VMEM((ch_a,), jnp.int32),
            pltpu.VMEM((ch_a, WB), jnp.float32),
            pltpu.VMEM((ch_b,), jnp.int32),
            pltpu.VMEM((ch_b,), jnp.int32),
            pltpu.VMEM((ch_b, WB), jnp.float32),
            pltpu.SemaphoreType.DMA,
        ],
    )
    def k(table_hbm, src_hbm, src2_hbm, dst_hbm, z_hbm, out_hbm, acc,
          sidx_a, didx_a, rows_a, sidx_b, didx_b, rows_b, sem):
        c = lax.axis_index("c")
        s = lax.axis_index("s")
        r0 = s * RPT

        for idx_hbm, blk_off, out_off, e_base, ch, nch, sidx, didx, rows in (
                (src_hbm, c * NP, c * NP, 0, ch_a, nch_a,
                 sidx_a, didx_a, rows_a),
                (src2_hbm, None, (2 + c) * NP, c * (E // NC), ch_b, nch_b,
                 sidx_b, didx_b, rows_b)):
            pltpu.sync_copy(z_hbm, acc.at[pl.ds(r0, RPT)])
            plsc.subcore_barrier()
            e0 = e_base + s * (nch * ch)

            def chunk(j, carry, ch=ch, e0=e0, blk_off=blk_off,
                      idx_hbm=idx_hbm, sidx=sidx, didx=didx, rows=rows):
                base = e0 + j * ch
                pltpu.sync_copy(idx_hbm.at[pl.ds(base, ch)], sidx)
                pltpu.sync_copy(dst_hbm.at[pl.ds(base, ch)], didx)
                if blk_off is not None:
                    for kk in range(ch // 16):
                        sl = pl.ds(kk * 16, 16)
                        sidx[sl] = sidx[sl] + blk_off
                pltpu.async_copy(table_hbm.at[sidx], rows, sem).wait()
                pltpu.sync_copy(rows, acc.at[didx], add=True)
                return carry

            lax.fori_loop(0, nch, chunk, 0)
            plsc.subcore_barrier()
            pltpu.sync_copy(acc.at[pl.ds(r0, RPT)],
                            out_hbm.at[pl.ds(out_off + r0, RPT)])
            plsc.subcore_barrier()

    return k(table3, src, src2, dst, zeros_blk)


def _sc_counts(tab16, cidx, dst, zeros_blk):
    ept = E // (NC * NS)
    ch = 40
    nch = ept // ch

    @functools.partial(
        pl.kernel,
        out_type=jax.ShapeDtypeStruct((NC * NP, WB), jnp.float32),
        mesh=_mesh(),
        scratch_types=[
            pltpu.VMEM_SHARED((NP, WB), jnp.float32),
            pltpu.VMEM((ch,), jnp.int32),
            pltpu.VMEM((ch,), jnp.int32),
            pltpu.VMEM((ch, WB), jnp.float32),
            pltpu.SemaphoreType.DMA,
        ],
    )
    def k(tab_hbm, cidx_hbm, dst_hbm, z_hbm, out_hbm, acc, sidx, didx, rows, sem):
        c = lax.axis_index("c")
        s = lax.axis_index("s")
        r0 = s * RPT
        pltpu.sync_copy(z_hbm, acc.at[pl.ds(r0, RPT)])
        plsc.subcore_barrier()
        e0 = (c * NS + s) * ept

        def chunk(j, carry):
            base = e0 + j * ch
            pltpu.sync_copy(cidx_hbm.at[pl.ds(base, ch)], sidx)
            pltpu.sync_copy(dst_hbm.at[pl.ds(base, ch)], didx)
            pltpu.async_copy(tab_hbm.at[sidx], rows, sem).wait()
            pltpu.sync_copy(rows, acc.at[didx], add=True)
            return carry

        lax.fori_loop(0, nch, chunk, 0)
        plsc.subcore_barrier()
        pltpu.sync_copy(acc.at[pl.ds(r0, RPT)],
                        out_hbm.at[pl.ds(c * NP + r0, RPT)])

    return k(tab16, cidx, dst, zeros_blk)


def _sc_pool(table3, batch_pad, zeros_blk):
    ch = 80

    @functools.partial(
        pl.kernel,
        out_type=jax.ShapeDtypeStruct((4 * BP, WB), jnp.float32),
        mesh=_mesh(),
        scratch_types=[
            pltpu.VMEM_SHARED((BP, WB), jnp.float32),
            pltpu.VMEM((ch,), jnp.int32),
            pltpu.VMEM((ch, WB), jnp.float32),
            pltpu.SemaphoreType.DMA,
        ],
    )
    def k(table_hbm, b_hbm, z_hbm, out_hbm, acc, didx, rows, sem):
        c = lax.axis_index("c")
        s = lax.axis_index("s")

        for blk_off, out_off, n0, rpt in (
                (c * NP, c * BP, 0, RPT),
                (2 * NP, (2 + c) * BP, c * (NP // 2), RPT // 2)):
            @pl.when(s == 0)
            def _():
                pltpu.sync_copy(z_hbm.at[pl.ds(0, BP)], acc)

            plsc.subcore_barrier()
            r0 = n0 + s * rpt

            def chunk(j, carry):
                base = r0 + j * ch
                pltpu.sync_copy(table_hbm.at[pl.ds(blk_off + base, ch)], rows)
                pltpu.sync_copy(b_hbm.at[pl.ds(base, ch)], didx)
                pltpu.sync_copy(rows, acc.at[didx], add=True)
                return carry

            lax.fori_loop(0, rpt // ch, chunk, 0)
            plsc.subcore_barrier()

            @pl.when(s == 0)
            def _():
                pltpu.sync_copy(acc, out_hbm.at[pl.ds(out_off, BP)])

            plsc.subcore_barrier()

    return k(table3, batch_pad, zeros_blk)


_BLK = 400
_NB = N // _BLK


def _tc_embed(x, cp, a1p, a2p):
    def body(x_ref, cp_ref, a1_ref, a2_ref, h_ref, c_ref):
        xa = x_ref[:, 0]
        xb = x_ref[:, 1]
        ii = lax.broadcasted_iota(jnp.int32, (_BLK, 16), 1)
        oh1 = (xa[:, None] == ii).astype(jnp.float32)
        oh2 = (xb[:, None] == ii).astype(jnp.float32)
        h = jnp.dot(oh1, a1_ref[...], preferred_element_type=jnp.float32, precision=lax.Precision.HIGHEST)
        h = h + jnp.dot(oh2, a2_ref[...], preferred_element_type=jnp.float32, precision=lax.Precision.HIGHEST)
        c_ref[...] = (cp_ref[0] + cp_ref[1])[:, :16]
        for b in range(3):
            h_ref[b] = h[:, b * WB:(b + 1) * WB]

    return pl.pallas_call(
        body,
        grid=(_NB,),
        in_specs=[
            pl.BlockSpec((_BLK, 2), lambda i: (i, 0)),
            pl.BlockSpec((2, _BLK, WB), lambda i: (0, i, 0)),
            pl.BlockSpec((16, W), lambda i: (0, 0)),
            pl.BlockSpec((16, W), lambda i: (0, 0)),
        ],
        out_specs=[
            pl.BlockSpec((3, _BLK, WB), lambda i: (0, i, 0)),
            pl.BlockSpec((_BLK, 16), lambda i: (i, 0)),
        ],
        out_shape=[
            jax.ShapeDtypeStruct((3, NP, WB), jnp.float32),
            jax.ShapeDtypeStruct((N, 16), jnp.float32),
        ],
    )(x, cp, a1p, a2p)


def _tc_layer(scout, h, cmat, wcp, slvp, w1p, b1p, w2p, b2p, gp, bp, last):
    def body(so_ref, h_ref, c_ref, wc_ref, slv_ref, w1_ref, b1_ref, w2_ref,
             b2_ref, g_ref, be_ref, o_ref, zbuf, stats):
        p = pl.program_id(0)
        i = pl.program_id(1)

        @pl.when(p == 0)
        def _():
            agg = jnp.concatenate(
                [so_ref[0] + h_ref[0], so_ref[1] + h_ref[1],
                 so_ref[2] + so_ref[3] + h_ref[2]], axis=1)
            agg = agg + jnp.dot(c_ref[...], wc_ref[...],
                                preferred_element_type=jnp.float32,
                                precision=lax.Precision.HIGHEST)
            agg = agg + slv_ref[...]
            z1 = jnp.maximum(
                jnp.dot(agg.astype(jnp.bfloat16),
                        w1_ref[...].astype(jnp.bfloat16),
                        preferred_element_type=jnp.float32) + b1_ref[...], 0.0)
            z = jnp.dot(z1.astype(jnp.bfloat16),
                        w2_ref[...].astype(jnp.bfloat16),
                        preferred_element_type=jnp.float32) + b2_ref[...]
            zbuf[pl.ds(i * _BLK, _BLK), :] = z

            @pl.when(i == 0)
            def _():
                stats[...] = jnp.zeros((8, W), jnp.float32)

            stats[0:1, :] += jnp.sum(z, axis=0, keepdims=True)

        nn = jnp.float32(N)

        @pl.when(p == 1)
        def _():
            mean = stats[0:1, :] / nn
            d = zbuf[pl.ds(i * _BLK, _BLK), :] - mean
            stats[1:2, :] += jnp.sum(d * d, axis=0, keepdims=True)

        @pl.when(p == 2)
        def _():
            mean = stats[0:1, :] / nn
            var = stats[1:2, :] / nn
            inv = lax.rsqrt(var + 1e-5)
            z = zbuf[pl.ds(i * _BLK, _BLK), :]
            hn = (z - mean) * (inv * g_ref[...]) + be_ref[...]
            if not last:
                hn = jnp.maximum(hn, 0.0)
            else:
                col = lax.broadcasted_iota(jnp.int32, (1, W), 1)
                hn = hn + (col == ONE_COL).astype(jnp.float32)
            for b in range(3):
                o_ref[b] = hn[:, b * WB:(b + 1) * WB]

    return pl.pallas_call(
        body,
        grid=(3, _NB),
        in_specs=[
            pl.BlockSpec((4, _BLK, WB), lambda p, i: (0, i, 0)),
            pl.BlockSpec((3, _BLK, WB), lambda p, i: (0, i, 0)),
            pl.BlockSpec((_BLK, 16), lambda p, i: (i, 0)),
            pl.BlockSpec((16, W), lambda p, i: (0, 0)),
            pl.BlockSpec((1, W), lambda p, i: (0, 0)),
            pl.BlockSpec((W, 2 * EMB), lambda p, i: (0, 0)),
            pl.BlockSpec((1, 2 * EMB), lambda p, i: (0, 0)),
            pl.BlockSpec((2 * EMB, W), lambda p, i: (0, 0)),
            pl.BlockSpec((1, W), lambda p, i: (0, 0)),
            pl.BlockSpec((1, W), lambda p, i: (0, 0)),
            pl.BlockSpec((1, W), lambda p, i: (0, 0)),
        ],
        out_specs=pl.BlockSpec((3, _BLK, WB), lambda p, i: (0, i, 0)),
        out_shape=jax.ShapeDtypeStruct((3, NP, WB), jnp.float32),
        scratch_shapes=[
            pltpu.VMEM((N, W), jnp.float32),
            pltpu.VMEM((8, W), jnp.float32),
        ],
    )(scout, h, cmat, wcp, slvp, w1p, b1p, w2p, b2p, gp, bp)


def _tc_proj(pool0, pool1, wp1p, bp1, wp2, bp2):
    def body(p0_ref, p1_ref, w1_ref, b1_ref, w2_ref, b2_ref, o_ref):
        def proj(p_ref):
            sfull = jnp.concatenate(
                [p_ref[0, :B, :], p_ref[1, :B, :],
                 p_ref[2, :B, :] + p_ref[3, :B, :]], axis=1)
            cnt = sfull[:, ONE_COL:ONE_COL + 1]
            g = sfull * (1.0 / jnp.maximum(cnt, 1.0))
            z1 = jnp.maximum(
                jnp.dot(g.astype(jnp.bfloat16),
                        w1_ref[...].astype(jnp.bfloat16),
                        preferred_element_type=jnp.float32) + b1_ref[...], 0.0)
            p = jnp.dot(z1.astype(jnp.bfloat16),
                        w2_ref[...].astype(jnp.bfloat16),
                        preferred_element_type=jnp.float32) + b2_ref[...]
            nrm = jnp.sqrt(jnp.sum(p * p, axis=1, keepdims=True))
            return p / jnp.maximum(nrm, 1e-12)

        f0 = proj(p0_ref)
        f1 = proj(p1_ref)
        o_ref[...] = lax.dot_general(
            f0.astype(jnp.bfloat16), f1.astype(jnp.bfloat16),
            (((1,), (1,)), ((), ())),
            preferred_element_type=jnp.float32) * (1.0 / TEMP)

    return pl.pallas_call(
        body,
        in_specs=[
            pl.BlockSpec((4, BP, WB), lambda: (0, 0, 0)),
            pl.BlockSpec((4, BP, WB), lambda: (0, 0, 0)),
            pl.BlockSpec((W, EMB), lambda: (0, 0)),
            pl.BlockSpec((1, EMB), lambda: (0, 0)),
            pl.BlockSpec((EMB, EMB), lambda: (0, 0)),
            pl.BlockSpec((1, EMB), lambda: (0, 0)),
        ],
        out_specs=pl.BlockSpec((B, B), lambda: (0, 0)),
        out_shape=jax.ShapeDtypeStruct((B, B), jnp.float32),
    )(pool0, pool1, wp1p, bp1, wp2, bp2)


def _padw(a, rows):
    out = jnp.zeros((rows, W), jnp.float32)
    return out.at[:a.shape[0], :a.shape[1]].set(a)


def kernel(x0, edge_index0, edge_attr0, batch0, x1, edge_index1, edge_attr1,
           batch1, params):
    f32 = jnp.float32
    i32 = jnp.int32

    a1p = _padw(params['atom_emb1'][:3, :].astype(f32), 16)
    a2p = _padw(params['atom_emb2'].astype(f32), 16)

    lw = []
    for l in range(LAYERS):
        lp = params['layers'][l]
        e1 = lp['edge_emb1'].astype(f32)
        e2 = lp['edge_emb2'].astype(f32)
        wc = jnp.zeros((16, W), f32)
        wc = wc.at[0:6, :EMB].set(e1).at[6:9, :EMB].set(e2)
        w1p = jnp.zeros((W, 2 * EMB), f32).at[:EMB, :].set(lp['W1'].astype(f32))
        slvp = _padw((e1[4] + e2[0])[None, :], 1)
        b1p = lp['b1'].astype(f32)[None, :]
        w2p = jnp.zeros((2 * EMB, W), f32).at[:, :EMB].set(lp['W2'].astype(f32))
        b2p = _padw(lp['b2'].astype(f32)[None, :], 1)
        gp = _padw(lp['gamma'].astype(f32)[None, :], 1)
        bp = _padw(lp['beta'].astype(f32)[None, :], 1)
        lw.append((wc, slvp, w1p, b1p, w2p, b2p, gp, bp))

    wp1p = jnp.zeros((W, EMB), f32).at[:EMB, :].set(params['Wp1'].astype(f32))
    bp1 = params['bp1'].astype(f32)[None, :]
    wp2 = params['Wp2'].astype(f32)
    bp2 = params['bp2'].astype(f32)[None, :]

    j = jnp.arange(16)
    tab16 = (((j[:, None] // 3 == j[None, :]) & (j[:, None] < 9)).astype(f32)
             + ((j[:, None] % 3 + 6 == j[None, :])
                & (j[:, None] < 9)).astype(f32))
    tab16 = jnp.concatenate([tab16, jnp.zeros((16, WB - 16), f32)], axis=1)

    zeros_blk = jnp.zeros((RPT, WB), f32)

    def encode(x, ei, ea, batch):
        src = ei[0].astype(i32)
        src2 = src + 2 * NP
        dst = ei[1].astype(i32)
        cidx = (ea[:, 0] * 3 + ea[:, 1]).astype(i32)
        bpad = jnp.concatenate(
            [batch.astype(i32), jnp.full((NP - N,), B, i32)])

        cp = _sc_counts(tab16, cidx, dst, zeros_blk)
        cp = cp.reshape(NC, NP, WB)[:, :N, :]
        h, cmat = _tc_embed(x.astype(i32), cp, a1p, a2p)
        for l in range(LAYERS):
            wc, slvp, w1p, b1p, w2p, b2p, gp, bp = lw[l]
            so = _sc_spmm(h.reshape(3 * NP, WB), src, src2, dst, zeros_blk)
            h = _tc_layer(so.reshape(4, NP, WB), h, cmat, wc, slvp, w1p, b1p,
                          w2p, b2p, gp, bp, last=(l == LAYERS - 1))
        pooled = _sc_pool(h.reshape(3 * NP, WB), bpad, zeros_blk)
        return pooled.reshape(4, BP, WB)

    pool0 = encode(x0, edge_index0, edge_attr0, batch0)
    pool1 = encode(x1, edge_index1, edge_attr1, batch1)
    logits = _tc_proj(pool0, pool1, wp1p, bp1, wp2, bp2)
    labels = jnp.arange(B, dtype=i32)
    return logits, labels
